# baseline (speedup 1.0000x reference)
import functools

import jax
import jax.numpy as jnp
from jax import lax
from jax.experimental import pallas as pl
from jax.experimental.pallas import tpu as pltpu
from jax.experimental.pallas import tpu_sc as plsc

N = 9216
K = 8192
D = 64
BR = 1152
GRID = N // BR
COMMIT_W = 0.25


def _argmin_body(x_ref, emb_ref, codes_ref, esq_ref):
    i = pl.program_id(0)
    emb = emb_ref[...]

    @pl.when(i == 0)
    def _prep():
        ones = jnp.ones((1, D), jnp.float32)
        esq_ref[...] = jax.lax.dot_general(
            ones, emb * emb, (((1,), (1,)), ((), ())),
            preferred_element_type=jnp.float32,
            precision=jax.lax.Precision.HIGHEST)

    xm2 = x_ref[...] * (-2.0)
    xe = jax.lax.dot_general(xm2, emb, (((1,), (1,)), ((), ())),
                             preferred_element_type=jnp.float32)
    d = xe + esq_ref[...]
    codes_ref[0, 0, :] = jnp.argmin(d, axis=1).astype(jnp.int32)


def _codes(x, embeddings):
    rows = x.shape[0]
    grid = rows // BR
    codes3 = pl.pallas_call(
        _argmin_body,
        grid=(grid,),
        in_specs=[
            pl.BlockSpec((BR, D), lambda i: (i, 0)),
            pl.BlockSpec((K, D), lambda i: (0, 0)),
        ],
        out_specs=pl.BlockSpec((1, 1, BR), lambda i: (i, 0, 0)),
        out_shape=jax.ShapeDtypeStruct((grid, 1, BR), jnp.int32),
        scratch_shapes=[pltpu.VMEM((1, K), jnp.float32)],
    )(x, embeddings)
    return codes3.reshape(rows)


DPAD = 128


def _make_sc_gather(rows):
    info = plsc.get_sparse_core_info()
    nw = info.num_cores * info.num_subcores
    b_per_w = rows // nw
    mesh = plsc.VectorSubcoreMesh(core_axis_name="c", subcore_axis_name="s")

    @functools.partial(
        pl.kernel, mesh=mesh,
        out_type=jax.ShapeDtypeStruct((rows, DPAD), jnp.float32),
        scratch_types=[
            pltpu.VMEM((b_per_w,), jnp.int32),
            pltpu.VMEM((b_per_w, DPAD), jnp.float32),
            pltpu.SemaphoreType.DMA,
        ],
    )
    def gather(table_hbm, idx_hbm, out_hbm, idx_v, rows_v, sem):
        wid = lax.axis_index("s") * info.num_cores + lax.axis_index("c")
        base = wid * b_per_w
        pltpu.sync_copy(idx_hbm.at[pl.ds(base, b_per_w)], idx_v)
        pltpu.async_copy(table_hbm.at[idx_v], rows_v, sem).wait()
        pltpu.sync_copy(rows_v, out_hbm.at[pl.ds(base, b_per_w)])

    return gather


def _epilogue_body(x_ref, wide_ref, ek_ref, resid_ref, loss_ref):
    e_k = wide_ref[:, :D]
    ek_ref[...] = e_k
    res = x_ref[...] - e_k
    resid_ref[...] = res
    loss_ref[...] = jnp.sum(res * res, keepdims=True)


def _epilogue(x, ek_wide):
    return pl.pallas_call(
        _epilogue_body,
        in_specs=[
            pl.BlockSpec((N, D), lambda: (0, 0)),
            pl.BlockSpec((N, DPAD), lambda: (0, 0)),
        ],
        out_specs=[
            pl.BlockSpec((N, D), lambda: (0, 0)),
            pl.BlockSpec((N, D), lambda: (0, 0)),
            pl.BlockSpec((1, 1), lambda: (0, 0)),
        ],
        out_shape=[
            jax.ShapeDtypeStruct((N, D), jnp.float32),
            jax.ShapeDtypeStruct((N, D), jnp.float32),
            jax.ShapeDtypeStruct((1, 1), jnp.float32),
        ],
    )(x, ek_wide)


def kernel(x, embeddings):
    table = jnp.pad(embeddings, ((0, 0), (0, DPAD - D)))
    half = N // 2
    gather = _make_sc_gather(half)
    codes_a = _codes(x[:half], embeddings)
    ek_a = gather(table, codes_a)
    codes_b = _codes(x[half:], embeddings)
    ek_b = gather(table, codes_b)
    codes = jnp.concatenate([codes_a, codes_b])
    ek_wide = jnp.concatenate([ek_a, ek_b])
    e_k, resid, loss = _epilogue(x, ek_wide)
    commitment = loss[0, 0] * ((1.0 + COMMIT_W) / (N * D))
    return codes, e_k, resid, commitment


# speedup vs baseline: 1.1296x; 1.1296x over previous
import functools

import jax
import jax.numpy as jnp
from jax import lax
from jax.experimental import pallas as pl
from jax.experimental.pallas import tpu as pltpu
from jax.experimental.pallas import tpu_sc as plsc

N = 9216
K = 8192
D = 64
BR = 1024
GRID = N // BR
COMMIT_W = 0.25


def _argmin_body(x_ref, emb_ref, codes_ref, esq_ref):
    i = pl.program_id(0)
    emb = emb_ref[...]

    @pl.when(i == 0)
    def _prep():
        ones = jnp.ones((1, D), jnp.float32)
        esq_ref[...] = jax.lax.dot_general(
            ones, emb * emb, (((1,), (1,)), ((), ())),
            preferred_element_type=jnp.float32,
            precision=jax.lax.Precision.HIGHEST)

    xm2 = x_ref[...] * (-2.0)
    xe = jax.lax.dot_general(xm2, emb, (((1,), (1,)), ((), ())),
                             preferred_element_type=jnp.float32)
    d = xe + esq_ref[...]
    codes_ref[0, 0, :] = jnp.argmin(d, axis=1).astype(jnp.int32)


def _codes(x, embeddings):
    rows = x.shape[0]
    grid = rows // BR
    codes3 = pl.pallas_call(
        _argmin_body,
        grid=(grid,),
        in_specs=[
            pl.BlockSpec((BR, D), lambda i: (i, 0)),
            pl.BlockSpec((K, D), lambda i: (0, 0)),
        ],
        out_specs=pl.BlockSpec((1, 1, BR), lambda i: (i, 0, 0)),
        out_shape=jax.ShapeDtypeStruct((grid, 1, BR), jnp.int32),
        scratch_shapes=[pltpu.VMEM((1, K), jnp.float32)],
    )(x, embeddings)
    return codes3.reshape(rows)


DPAD = 128


def _make_sc_gather(rows):
    info = plsc.get_sparse_core_info()
    nw = info.num_cores * info.num_subcores
    b_per_w = rows // nw
    mesh = plsc.VectorSubcoreMesh(core_axis_name="c", subcore_axis_name="s")

    @functools.partial(
        pl.kernel, mesh=mesh,
        out_type=jax.ShapeDtypeStruct((rows, DPAD), jnp.float32),
        scratch_types=[
            pltpu.VMEM((b_per_w,), jnp.int32),
            pltpu.VMEM((b_per_w, DPAD), jnp.float32),
            pltpu.SemaphoreType.DMA,
        ],
    )
    def gather(table_hbm, idx_hbm, out_hbm, idx_v, rows_v, sem):
        wid = lax.axis_index("s") * info.num_cores + lax.axis_index("c")
        base = wid * b_per_w
        pltpu.sync_copy(idx_hbm.at[pl.ds(base, b_per_w)], idx_v)
        pltpu.async_copy(table_hbm.at[idx_v], rows_v, sem).wait()
        pltpu.sync_copy(rows_v, out_hbm.at[pl.ds(base, b_per_w)])

    return gather


def _epilogue_body(x_ref, wide_ref, ek_ref, resid_ref, loss_ref):
    e_k = wide_ref[:, :D]
    ek_ref[...] = e_k
    res = x_ref[...] - e_k
    resid_ref[...] = res
    loss_ref[...] = jnp.sum(res * res, keepdims=True)


def _epilogue(x, ek_wide):
    return pl.pallas_call(
        _epilogue_body,
        in_specs=[
            pl.BlockSpec((N, D), lambda: (0, 0)),
            pl.BlockSpec((N, DPAD), lambda: (0, 0)),
        ],
        out_specs=[
            pl.BlockSpec((N, D), lambda: (0, 0)),
            pl.BlockSpec((N, D), lambda: (0, 0)),
            pl.BlockSpec((1, 1), lambda: (0, 0)),
        ],
        out_shape=[
            jax.ShapeDtypeStruct((N, D), jnp.float32),
            jax.ShapeDtypeStruct((N, D), jnp.float32),
            jax.ShapeDtypeStruct((1, 1), jnp.float32),
        ],
    )(x, ek_wide)


def kernel(x, embeddings):
    table = jnp.pad(embeddings, ((0, 0), (0, DPAD - D)))
    codes = _codes(x, embeddings)
    ek_wide = _make_sc_gather(N)(table, codes)
    e_k, resid, loss = _epilogue(x, ek_wide)
    commitment = loss[0, 0] * ((1.0 + COMMIT_W) / (N * D))
    return codes, e_k, resid, commitment
